# baseline (speedup 1.0000x reference)
import jax
import jax.numpy as jnp
from jax import lax
from jax.experimental import pallas as pl
from jax.experimental.pallas import tpu as pltpu
from jax.experimental.pallas import tpu_sc as plsc

_T = 16384
_E = 64
_NC, _NS, _L = 2, 16, 16
_NW = _NC * _NS
_RPW = _T // _NW
_NB = 8
_RPB = _T // _NB


def _tc1_body(q_ref, idx_ref, cs_ref):
    v = q_ref[...]
    rm = jnp.max(v, axis=1, keepdims=True)
    pos = jax.lax.broadcasted_iota(jnp.int32, v.shape, 1)
    elig = jnp.where(v == rm, pos, _E)
    idx_ref[...] = jnp.min(elig, axis=1).reshape(1, 1, _RPB)
    cs_ref[...] = jnp.sum(v, axis=0).reshape(1, 1, _E)


def _sc_body(idx_hbm, hist_hbm, ibuf, obuf):
    wid = lax.axis_index("s") * _NC + lax.axis_index("c")
    blk = wid // (_RPB // _RPW)
    off = (wid % (_RPB // _RPW)) * _RPW
    pltpu.sync_copy(idx_hbm.at[blk, 0, pl.ds(off, _RPW)], ibuf)

    zf = jnp.zeros((_L,), jnp.float32)
    for j in range(_E // _L):
        obuf[pl.ds(j * _L, _L)] = zf
    ones = jnp.ones((_L,), jnp.float32)

    def g_body(g, carry):
        a = ibuf[pl.ds(g * _L, _L)]
        plsc.addupdate_scatter(obuf, [a], ones)
        return carry

    lax.fori_loop(0, _RPW // _L, g_body, 0)
    pltpu.sync_copy(obuf, hist_hbm.at[wid])


_sc_call = pl.kernel(
    _sc_body,
    out_type=jax.ShapeDtypeStruct((_NW, _E), jnp.float32),
    mesh=plsc.VectorSubcoreMesh(core_axis_name="c", subcore_axis_name="s"),
    compiler_params=pltpu.CompilerParams(needs_layout_passes=False),
    scratch_types=[
        pltpu.VMEM((_RPW,), jnp.int32),
        pltpu.VMEM((_E,), jnp.float32),
    ],
)


def _tc2_body(cs_ref, h_ref, o_ref):
    cs = jnp.sum(cs_ref[...], axis=(0, 1))
    ct = jnp.sum(h_ref[...], axis=0)
    o_ref[...] = (jnp.sum(cs * ct) * (_E / (_T * _T))).reshape(1, 1)


def kernel(q):
    idx, cs = pl.pallas_call(
        _tc1_body,
        grid=(_NB,),
        in_specs=[pl.BlockSpec((_RPB, _E), lambda b: (b, 0))],
        out_specs=[
            pl.BlockSpec((1, 1, _RPB), lambda b: (b, 0, 0)),
            pl.BlockSpec((1, 1, _E), lambda b: (b, 0, 0)),
        ],
        out_shape=[
            jax.ShapeDtypeStruct((_NB, 1, _RPB), jnp.int32),
            jax.ShapeDtypeStruct((_NB, 1, _E), jnp.float32),
        ],
    )(q)
    hist = _sc_call(idx)
    out = pl.pallas_call(
        _tc2_body,
        out_shape=jax.ShapeDtypeStruct((1, 1), jnp.float32),
    )(cs, hist)
    return out[0, 0]


# speedup vs baseline: 1.4112x; 1.4112x over previous
import jax
import jax.numpy as jnp
from jax import lax
from jax.experimental import pallas as pl
from jax.experimental.pallas import tpu as pltpu
from jax.experimental.pallas import tpu_sc as plsc

_T = 16384
_E = 64
_NC, _NS, _L = 2, 16, 16
_NW = _NC * _NS
_RPW = _T // _NW
_G = _RPW // _L
_EB = _E // _L


def _sc_body(qt_hbm, acc_hbm, hist_hbm, chunk, m_ref, a_ref, obuf1, obuf2):
    wid = lax.axis_index("s") * _NC + lax.axis_index("c")
    base = wid * _RPW
    pltpu.sync_copy(qt_hbm.at[:, pl.ds(base, _RPW)], chunk)

    zf = jnp.zeros((_L,), jnp.float32)
    zi = jnp.zeros((_L,), jnp.int32)
    ones = jnp.ones((_L,), jnp.float32)

    for eb in range(_EB):
        def g_body(g, accs, eb=eb):
            accs = list(accs)
            if eb == 0:
                m = chunk[0, pl.ds(g * _L, _L)]
                a = zi
                accs[0] = accs[0] + m
                lo = 1
            else:
                m = m_ref[g]
                a = a_ref[g]
                lo = 0
            for j in range(lo, _L):
                e = eb * _L + j
                v = chunk[e, pl.ds(g * _L, _L)]
                accs[j] = accs[j] + v
                gt = v > m
                m = jnp.where(gt, v, m)
                a = jnp.where(gt, e, a)
            m_ref[g] = m
            a_ref[g] = a
            return tuple(accs)

        accs = lax.fori_loop(0, _G, g_body, (zf,) * _L)
        for j in range(_L):
            obuf1[eb * _L + j] = accs[j]

    for j in range(_EB):
        obuf2[pl.ds(j * _L, _L)] = zf

    def h_body(g, carry):
        plsc.addupdate_scatter(obuf2, [a_ref[g]], ones)
        return carry

    lax.fori_loop(0, _G, h_body, 0)

    pltpu.sync_copy(obuf1, acc_hbm.at[wid])
    pltpu.sync_copy(obuf2, hist_hbm.at[wid])


_sc_call = pl.kernel(
    _sc_body,
    out_type=[
        jax.ShapeDtypeStruct((_NW, _E, _L), jnp.float32),
        jax.ShapeDtypeStruct((_NW, _E), jnp.float32),
    ],
    mesh=plsc.VectorSubcoreMesh(core_axis_name="c", subcore_axis_name="s"),
    compiler_params=pltpu.CompilerParams(needs_layout_passes=False),
    scratch_types=[
        pltpu.VMEM((_E, _RPW), jnp.float32),
        pltpu.VMEM((_G, _L), jnp.float32),
        pltpu.VMEM((_G, _L), jnp.int32),
        pltpu.VMEM((_E, _L), jnp.float32),
        pltpu.VMEM((_E,), jnp.float32),
    ],
)


def _tc_body(a_ref, h_ref, o_ref):
    cs = jnp.sum(jnp.sum(a_ref[...], axis=0), axis=1)
    ct = jnp.sum(h_ref[...], axis=0)
    o_ref[...] = (jnp.sum(cs * ct) * (_E / (_T * _T))).reshape(1, 1)


def kernel(q):
    acc, hist = _sc_call(q.T)
    out = pl.pallas_call(
        _tc_body,
        out_shape=jax.ShapeDtypeStruct((1, 1), jnp.float32),
    )(acc, hist)
    return out[0, 0]


# speedup vs baseline: 1.4565x; 1.0321x over previous
import jax
import jax.numpy as jnp
from jax import lax
from jax.experimental import pallas as pl
from jax.experimental.pallas import tpu as pltpu
from jax.experimental.pallas import tpu_sc as plsc

_T = 16384
_E = 64
_NC, _NS, _L = 2, 16, 16
_NW = _NC * _NS
_RPW = _T // _NW
_G = _RPW // _L
_EB = _E // _L


def _sc_body(qt_hbm, acc_hbm, hist_hbm, chunk, m_ref, a_ref, obuf1, obuf2):
    wid = lax.axis_index("s") * _NC + lax.axis_index("c")
    base = wid * _RPW
    pltpu.sync_copy(qt_hbm.at[:, pl.ds(base, _RPW)], chunk)

    zf = jnp.zeros((_L,), jnp.float32)
    zi = jnp.zeros((_L,), jnp.int32)
    ones = jnp.ones((_L,), jnp.float32)

    for eb in range(_EB):
        def g_body(g, accs, eb=eb):
            accs = list(accs)
            v = [chunk[eb * _L + j, pl.ds(g * _L, _L)] for j in range(_L)]
            for j in range(_L):
                accs[j] = accs[j] + v[j]
            mt = list(v)
            at = [jnp.full((_L,), eb * _L + j, jnp.int32) for j in range(_L)]
            n = 1
            while n < _L:
                for j in range(0, _L, 2 * n):
                    ge = mt[j] >= mt[j + n]
                    mt[j] = jnp.where(ge, mt[j], mt[j + n])
                    at[j] = jnp.where(ge, at[j], at[j + n])
                n *= 2
            if eb == 0:
                m, a = mt[0], at[0]
            else:
                gt = mt[0] > m_ref[g]
                m = jnp.where(gt, mt[0], m_ref[g])
                a = jnp.where(gt, at[0], a_ref[g])
            m_ref[g] = m
            a_ref[g] = a
            return tuple(accs)

        accs = lax.fori_loop(0, _G, g_body, (zf,) * _L)
        for j in range(_L):
            obuf1[eb * _L + j] = accs[j]

    for j in range(_EB):
        obuf2[pl.ds(j * _L, _L)] = zf

    def h_body(g, carry):
        plsc.addupdate_scatter(obuf2, [a_ref[g]], ones)
        return carry

    lax.fori_loop(0, _G, h_body, 0)

    pltpu.sync_copy(obuf1, acc_hbm.at[wid])
    pltpu.sync_copy(obuf2, hist_hbm.at[wid])


_sc_call = pl.kernel(
    _sc_body,
    out_type=[
        jax.ShapeDtypeStruct((_NW, _E, _L), jnp.float32),
        jax.ShapeDtypeStruct((_NW, _E), jnp.float32),
    ],
    mesh=plsc.VectorSubcoreMesh(core_axis_name="c", subcore_axis_name="s"),
    compiler_params=pltpu.CompilerParams(needs_layout_passes=False),
    scratch_types=[
        pltpu.VMEM((_E, _RPW), jnp.float32),
        pltpu.VMEM((_G, _L), jnp.float32),
        pltpu.VMEM((_G, _L), jnp.int32),
        pltpu.VMEM((_E, _L), jnp.float32),
        pltpu.VMEM((_E,), jnp.float32),
    ],
)


def _tc_body(a_ref, h_ref, o_ref):
    cs = jnp.sum(jnp.sum(a_ref[...], axis=0), axis=1)
    ct = jnp.sum(h_ref[...], axis=0)
    o_ref[...] = (jnp.sum(cs * ct) * (_E / (_T * _T))).reshape(1, 1)


def kernel(q):
    acc, hist = _sc_call(q.T)
    out = pl.pallas_call(
        _tc_body,
        out_shape=jax.ShapeDtypeStruct((1, 1), jnp.float32),
    )(acc, hist)
    return out[0, 0]
